# speedup vs baseline: 5.1645x; 5.1645x over previous
import functools

import jax
import jax.numpy as jnp
from jax import lax
from jax.experimental import pallas as pl
from jax.experimental.pallas import tpu as pltpu
from jax.experimental.pallas import tpu_sc as plsc

NUM_CORES = 2
NUM_SUBCORES = 16
NUM_WORKERS = NUM_CORES * NUM_SUBCORES
CHUNK = 128


@functools.partial(jax.jit, static_argnums=(2, 3))
def _gather(flat_ids, weight, b, d):
    b_per_w = b // NUM_WORKERS
    chunks_per_w = b_per_w // CHUNK
    mesh = plsc.VectorSubcoreMesh(core_axis_name="c", subcore_axis_name="s")

    @functools.partial(
        pl.kernel,
        out_type=jax.ShapeDtypeStruct((b, d), jnp.float32),
        mesh=mesh,
        scratch_types=[
            pltpu.VMEM((CHUNK,), jnp.int32),
            pltpu.VMEM((CHUNK, d), jnp.float32),
            pltpu.SemaphoreType.DMA,
        ],
    )
    def k(idx_hbm, table_hbm, out_hbm, idx_v, rows_v, sem):
        wid = lax.axis_index("s") * NUM_CORES + lax.axis_index("c")
        base = wid * b_per_w

        @pl.loop(0, chunks_per_w)
        def _(j):
            off = base + j * CHUNK
            pltpu.sync_copy(idx_hbm.at[pl.ds(off, CHUNK)], idx_v)
            pltpu.async_copy(table_hbm.at[idx_v], rows_v, sem).wait()
            pltpu.sync_copy(rows_v, out_hbm.at[pl.ds(off, CHUNK)])

    return k(flat_ids, weight)


def kernel(token_ids, weight):
    s, t = token_ids.shape
    n, d = weight.shape
    flat = token_ids.reshape(s * t).astype(jnp.int32)
    out = _gather(flat, weight, s * t, d)
    return out.reshape(s, t, d)


# speedup vs baseline: 9.2042x; 1.7822x over previous
import functools

import jax
import jax.numpy as jnp
from jax import lax
from jax.experimental import pallas as pl
from jax.experimental.pallas import tpu as pltpu
from jax.experimental.pallas import tpu_sc as plsc

NUM_CORES = 2
NUM_SUBCORES = 16
NUM_WORKERS = NUM_CORES * NUM_SUBCORES
CHUNK = 128
NBUF = 4


@functools.partial(jax.jit, static_argnums=(2, 3))
def _gather(ids2d, weight, b, d):
    b_per_w = b // NUM_WORKERS
    chunks_per_w = b_per_w // CHUNK
    nouter = chunks_per_w // NBUF
    mesh = plsc.VectorSubcoreMesh(core_axis_name="c", subcore_axis_name="s")

    @functools.partial(
        pl.kernel,
        out_type=jax.ShapeDtypeStruct((b, d), jnp.float32),
        mesh=mesh,
        scratch_types=[
            pltpu.VMEM((chunks_per_w, CHUNK), jnp.int32),
            pltpu.VMEM((NBUF, CHUNK, d), jnp.float32),
            pltpu.SemaphoreType.DMA,
            pltpu.SemaphoreType.DMA,
        ],
    )
    def k(idx_hbm, table_hbm, out_hbm, idx_v, rows_v, gsem, ssem):
        wid = lax.axis_index("s") * NUM_CORES + lax.axis_index("c")
        cbase = wid * chunks_per_w
        rbase = wid * b_per_w

        pltpu.sync_copy(idx_hbm.at[pl.ds(cbase, chunks_per_w)], idx_v)

        def start_gather(buf, j):
            pltpu.async_copy(table_hbm.at[idx_v.at[j]], rows_v.at[buf], gsem)

        def wait_gather(buf):
            pltpu.make_async_copy(
                table_hbm.at[idx_v.at[0]], rows_v.at[buf], gsem
            ).wait()

        def start_store(buf, j):
            pltpu.async_copy(
                rows_v.at[buf], out_hbm.at[pl.ds(rbase + j * CHUNK, CHUNK)], ssem
            )

        def wait_store(buf):
            pltpu.make_async_copy(
                rows_v.at[buf], out_hbm.at[pl.ds(rbase, CHUNK)], ssem
            ).wait()

        for buf in range(NBUF):
            start_gather(buf, buf)

        @pl.loop(0, nouter)
        def _(t):
            for buf in range(NBUF):
                wait_gather(buf)
                start_store(buf, t * NBUF + buf)
            for buf in range(NBUF):
                wait_store(buf)

                @pl.when(t < nouter - 1)
                def _():
                    start_gather(buf, (t + 1) * NBUF + buf)

    return k(ids2d, weight)


def kernel(token_ids, weight):
    s, t = token_ids.shape
    n, d = weight.shape
    b = s * t
    ids2d = token_ids.reshape(b // CHUNK, CHUNK).astype(jnp.int32)
    out = _gather(ids2d, weight, b, d)
    return out.reshape(s, t, d)


# speedup vs baseline: 9.2184x; 1.0015x over previous
import functools

import jax
import jax.numpy as jnp
from jax import lax
from jax.experimental import pallas as pl
from jax.experimental.pallas import tpu as pltpu
from jax.experimental.pallas import tpu_sc as plsc

NUM_CORES = 2
NUM_SUBCORES = 16
NUM_WORKERS = NUM_CORES * NUM_SUBCORES
CHUNK = 128
NBUF = 5


@functools.partial(jax.jit, static_argnums=(2, 3))
def _gather(ids2d, weight, b, d):
    b_per_w = b // NUM_WORKERS
    chunks_per_w = b_per_w // CHUNK
    nouter = chunks_per_w // NBUF
    mesh = plsc.VectorSubcoreMesh(core_axis_name="c", subcore_axis_name="s")

    @functools.partial(
        pl.kernel,
        out_type=jax.ShapeDtypeStruct((b, d), jnp.float32),
        mesh=mesh,
        scratch_types=[
            pltpu.VMEM((chunks_per_w, CHUNK), jnp.int32),
            pltpu.VMEM((NBUF, CHUNK, d), jnp.float32),
            pltpu.SemaphoreType.DMA,
            pltpu.SemaphoreType.DMA,
        ],
    )
    def k(idx_hbm, table_hbm, out_hbm, idx_v, rows_v, gsem, ssem):
        wid = lax.axis_index("s") * NUM_CORES + lax.axis_index("c")
        cbase = wid * chunks_per_w
        rbase = wid * b_per_w

        pltpu.sync_copy(idx_hbm.at[pl.ds(cbase, chunks_per_w)], idx_v)

        def start_gather(buf, j):
            pltpu.async_copy(table_hbm.at[idx_v.at[j]], rows_v.at[buf], gsem)

        def wait_gather(buf):
            pltpu.make_async_copy(
                table_hbm.at[idx_v.at[0]], rows_v.at[buf], gsem
            ).wait()

        def start_store(buf, j):
            pltpu.async_copy(
                rows_v.at[buf], out_hbm.at[pl.ds(rbase + j * CHUNK, CHUNK)], ssem
            )

        def wait_store(buf):
            pltpu.make_async_copy(
                rows_v.at[buf], out_hbm.at[pl.ds(rbase, CHUNK)], ssem
            ).wait()

        for buf in range(NBUF):
            start_gather(buf, buf)

        @pl.loop(0, nouter)
        def _(t):
            for buf in range(NBUF):
                wait_gather(buf)
                start_store(buf, t * NBUF + buf)
            for buf in range(NBUF):
                wait_store(buf)

                @pl.when(t < nouter - 1)
                def _():
                    start_gather(buf, (t + 1) * NBUF + buf)

    return k(ids2d, weight)


def kernel(token_ids, weight):
    s, t = token_ids.shape
    n, d = weight.shape
    b = s * t
    ids2d = token_ids.reshape(b // CHUNK, CHUNK).astype(jnp.int32)
    out = _gather(ids2d, weight, b, d)
    return out.reshape(s, t, d)
